# Initial kernel scaffold; baseline (speedup 1.0000x reference)
import jax
import jax.numpy as jnp
from jax.experimental import pallas as pl


def kernel(x, edge_index, W1l, b1l, W1r, W2l, b2l, W2r):
    raise NotImplementedError("write your pallas kernel here")


# speedup vs baseline: 5.6670x; 5.6670x over previous
import functools

import jax
import jax.numpy as jnp
from jax import lax
from jax.experimental import pallas as pl
from jax.experimental.pallas import tpu as pltpu
from jax.experimental.pallas import tpu_sc as plsc

N_NODES = 10000
D = 128
N_EDGES = 320000

N_PAD = 10240
E_PAD = 327680
EDGES_PER_TILE = E_PAD // 32
ROWS_PER_TILE = N_PAD // 16
CHUNK = 128
N_CHUNKS = EDGES_PER_TILE // CHUNK
N_WB = ROWS_PER_TILE // 128

_sc_mesh = plsc.VectorSubcoreMesh(core_axis_name="c", subcore_axis_name="s")


@functools.partial(
    pl.kernel, mesh=_sc_mesh,
    out_type=jax.ShapeDtypeStruct((2 * N_PAD, D), jnp.float32),
    scratch_types=[
        pltpu.VMEM((CHUNK,), jnp.int32),
        pltpu.VMEM((CHUNK,), jnp.int32),
        pltpu.VMEM((CHUNK, D), jnp.float32),
        pltpu.VMEM_SHARED((N_PAD, D), jnp.float32),
        pltpu.SemaphoreType.DMA,
    ],
)
def _aggregate(x_hbm, src_hbm, dst_hbm, zf, sum_out,
               src_v, dst_v, rows_v, acc_sh, sem):
    c = lax.axis_index("c")
    s = lax.axis_index("s")
    tile = c * 16 + s

    rb = s * ROWS_PER_TILE
    pltpu.sync_copy(zf, rows_v)
    for t in range(N_WB):
        pltpu.sync_copy(rows_v, acc_sh.at[pl.ds(rb + t * 128, 128)])
    plsc.subcore_barrier()

    edge_base = tile * EDGES_PER_TILE

    def chunk_body(k, carry):
        e = edge_base + k * CHUNK
        pltpu.sync_copy(src_hbm.at[pl.ds(e, CHUNK)], src_v)
        pltpu.sync_copy(dst_hbm.at[pl.ds(e, CHUNK)], dst_v)
        pltpu.async_copy(x_hbm.at[src_v], rows_v, sem).wait()
        pltpu.sync_copy(rows_v, acc_sh.at[dst_v], add=True)
        return carry

    lax.fori_loop(0, N_CHUNKS, chunk_body, 0)

    plsc.subcore_barrier()

    ob = c * N_PAD + s * ROWS_PER_TILE
    for t in range(N_WB):
        pltpu.sync_copy(acc_sh.at[pl.ds(rb + t * 128, 128)], rows_v)
        pltpu.sync_copy(rows_v, sum_out.at[pl.ds(ob + t * 128, 128)])


@functools.partial(
    pl.kernel, mesh=_sc_mesh,
    out_type=jax.ShapeDtypeStruct((2 * N_PAD, D), jnp.float32),
    scratch_types=[
        pltpu.VMEM((CHUNK,), jnp.int32),
        pltpu.VMEM((CHUNK, D), jnp.float32),
        pltpu.VMEM_SHARED((N_PAD, D), jnp.float32),
        pltpu.SemaphoreType.DMA,
    ],
)
def _degrees(dst_hbm, zf, ones_hbm, cnt_out, dst_v, ones_v, acc_sh, sem):
    c = lax.axis_index("c")
    s = lax.axis_index("s")
    tile = c * 16 + s

    rb = s * ROWS_PER_TILE
    pltpu.sync_copy(zf, ones_v)
    for t in range(N_WB):
        pltpu.sync_copy(ones_v, acc_sh.at[pl.ds(rb + t * 128, 128)])
    pltpu.sync_copy(ones_hbm, ones_v)
    plsc.subcore_barrier()

    edge_base = tile * EDGES_PER_TILE

    def chunk_body(k, carry):
        e = edge_base + k * CHUNK
        pltpu.sync_copy(dst_hbm.at[pl.ds(e, CHUNK)], dst_v)
        pltpu.sync_copy(ones_v, acc_sh.at[dst_v], add=True)
        return carry

    lax.fori_loop(0, N_CHUNKS, chunk_body, 0)

    plsc.subcore_barrier()

    ob = c * N_PAD + s * ROWS_PER_TILE
    for t in range(N_WB):
        pltpu.sync_copy(acc_sh.at[pl.ds(rb + t * 128, 128)], ones_v)
        pltpu.sync_copy(ones_v, cnt_out.at[pl.ds(ob + t * 128, 128)])


BLK = 1024


def _combine_body(relu, p_ref, cnt_ref, x_ref, wl_ref, wr_ref, b_ref, o_ref):
    ssum = p_ref[0] + p_ref[1]
    cnt = cnt_ref[0, :, 0] + cnt_ref[1, :, 0]
    agg = ssum / jnp.clip(cnt, 1.0, None)[:, None]
    out = (jnp.dot(agg, wl_ref[...], preferred_element_type=jnp.float32)
           + jnp.dot(x_ref[...], wr_ref[...], preferred_element_type=jnp.float32)
           + b_ref[0:1, :])
    if relu:
        out = jnp.maximum(out, 0.0)
    o_ref[...] = out


def _combine(p, cnt, x, wlT, wrT, b, relu):
    return pl.pallas_call(
        functools.partial(_combine_body, relu),
        grid=(N_PAD // BLK,),
        in_specs=[
            pl.BlockSpec((2, BLK, D), lambda i: (0, i, 0)),
            pl.BlockSpec((2, BLK, D), lambda i: (0, i, 0)),
            pl.BlockSpec((BLK, D), lambda i: (i, 0)),
            pl.BlockSpec((D, D), lambda i: (0, 0)),
            pl.BlockSpec((D, D), lambda i: (0, 0)),
            pl.BlockSpec((8, D), lambda i: (0, 0)),
        ],
        out_specs=pl.BlockSpec((BLK, D), lambda i: (i, 0)),
        out_shape=jax.ShapeDtypeStruct((N_PAD, D), jnp.float32),
    )(p, cnt, x, wlT, wrT, b)


def kernel(x, edge_index, W1l, b1l, W1r, W2l, b2l, W2r):
    src = edge_index[0].astype(jnp.int32)
    dst = edge_index[1].astype(jnp.int32)

    n_pad_e = E_PAD - N_EDGES
    pad_src = jnp.arange(n_pad_e, dtype=jnp.int32) % N_NODES
    pad_dst = N_NODES + (jnp.arange(n_pad_e, dtype=jnp.int32) % (N_PAD - N_NODES))
    srcr = jnp.concatenate([src, pad_src])
    dstr = jnp.concatenate([dst, pad_dst])

    zf = jnp.zeros((128, D), jnp.float32)
    ones = jnp.ones((128, D), jnp.float32)
    x_pad = jnp.concatenate([x, jnp.zeros((N_PAD - N_NODES, D), jnp.float32)])

    b1 = jnp.broadcast_to(b1l.reshape(1, D), (8, D))
    b2 = jnp.broadcast_to(b2l.reshape(1, D), (8, D))

    cnt = _degrees(dstr, zf, ones).reshape(2, N_PAD, D)
    p1 = _aggregate(x, srcr, dstr, zf).reshape(2, N_PAD, D)
    h = _combine(p1, cnt, x_pad, W1l.T, W1r.T, b1, relu=True)
    p2 = _aggregate(h, srcr, dstr, zf).reshape(2, N_PAD, D)
    out = _combine(p2, cnt, h, W2l.T, W2r.T, b2, relu=False)
    return out[:N_NODES]


# speedup vs baseline: 10.3964x; 1.8346x over previous
import functools

import jax
import jax.numpy as jnp
from jax import lax
from jax.experimental import pallas as pl
from jax.experimental.pallas import tpu as pltpu
from jax.experimental.pallas import tpu_sc as plsc

N_NODES = 10000
D = 128
N_EDGES = 320000

N_PAD = 10240
E_PAD = 327680
EDGES_PER_TILE = E_PAD // 32
ROWS_PER_TILE = N_PAD // 16
CHUNK = 128
N_CHUNKS = EDGES_PER_TILE // CHUNK
N_WB = ROWS_PER_TILE // 128
PHASES = 2
SLAB = N_CHUNKS // PHASES
PAIRS = SLAB // 2

_sc_mesh = plsc.VectorSubcoreMesh(core_axis_name="c", subcore_axis_name="s")


@functools.partial(
    pl.kernel, mesh=_sc_mesh,
    out_type=jax.ShapeDtypeStruct((2 * N_PAD, D), jnp.float32),
    scratch_types=[
        pltpu.VMEM((SLAB, CHUNK), jnp.int32),
        pltpu.VMEM((SLAB, CHUNK), jnp.int32),
        pltpu.VMEM((CHUNK, D), jnp.float32),
        pltpu.VMEM((CHUNK, D), jnp.float32),
        pltpu.VMEM_SHARED((N_PAD, D), jnp.float32),
        pltpu.SemaphoreType.DMA,
        pltpu.SemaphoreType.DMA,
    ],
)
def _aggregate(x_hbm, src_hbm, dst_hbm, zf, sum_out,
               src_v, dst_v, rows_a, rows_b, acc_sh, sem_a, sem_b):
    c = lax.axis_index("c")
    s = lax.axis_index("s")
    tile = c * 16 + s

    rb = s * ROWS_PER_TILE
    pltpu.sync_copy(zf, rows_a)
    for t in range(N_WB):
        pltpu.sync_copy(rows_a, acc_sh.at[pl.ds(rb + t * 128, 128)])
    plsc.subcore_barrier()

    for p in range(PHASES):
        slab_row = tile * N_CHUNKS + p * SLAB
        pltpu.sync_copy(src_hbm.at[pl.ds(slab_row, SLAB)], src_v)
        pltpu.sync_copy(dst_hbm.at[pl.ds(slab_row, SLAB)], dst_v)

        pltpu.async_copy(x_hbm.at[src_v.at[0]], rows_a, sem_a)

        def pair_body(k2, carry):
            ka = 2 * k2
            pltpu.async_copy(x_hbm.at[src_v.at[ka + 1]], rows_b, sem_b)
            pltpu.make_async_copy(x_hbm.at[src_v.at[ka]], rows_a, sem_a).wait()
            pltpu.sync_copy(rows_a, acc_sh.at[dst_v.at[ka]], add=True)

            @pl.when(k2 < PAIRS - 1)
            def _():
                pltpu.async_copy(x_hbm.at[src_v.at[ka + 2]], rows_a, sem_a)

            pltpu.make_async_copy(x_hbm.at[src_v.at[ka + 1]], rows_b, sem_b).wait()
            pltpu.sync_copy(rows_b, acc_sh.at[dst_v.at[ka + 1]], add=True)
            return carry

        lax.fori_loop(0, PAIRS, pair_body, 0)

    plsc.subcore_barrier()

    ob = c * N_PAD + s * ROWS_PER_TILE
    for t in range(N_WB):
        pltpu.sync_copy(acc_sh.at[pl.ds(rb + t * 128, 128)], rows_a)
        pltpu.sync_copy(rows_a, sum_out.at[pl.ds(ob + t * 128, 128)])


@functools.partial(
    pl.kernel, mesh=_sc_mesh,
    out_type=jax.ShapeDtypeStruct((2 * N_PAD, D), jnp.float32),
    scratch_types=[
        pltpu.VMEM((N_CHUNKS, CHUNK), jnp.int32),
        pltpu.VMEM((CHUNK, D), jnp.float32),
        pltpu.VMEM_SHARED((N_PAD, D), jnp.float32),
        pltpu.SemaphoreType.DMA,
    ],
)
def _degrees(dst_hbm, zf, ones_hbm, cnt_out, dst_v, ones_v, acc_sh, sem):
    c = lax.axis_index("c")
    s = lax.axis_index("s")
    tile = c * 16 + s

    rb = s * ROWS_PER_TILE
    pltpu.sync_copy(zf, ones_v)
    for t in range(N_WB):
        pltpu.sync_copy(ones_v, acc_sh.at[pl.ds(rb + t * 128, 128)])
    pltpu.sync_copy(ones_hbm, ones_v)
    pltpu.sync_copy(dst_hbm.at[pl.ds(tile * N_CHUNKS, N_CHUNKS)], dst_v)
    plsc.subcore_barrier()

    def chunk_body(k, carry):
        pltpu.async_copy(ones_v, acc_sh.at[dst_v.at[k]], sem, add=True)

        @pl.when(k > 0)
        def _():
            pltpu.make_async_copy(ones_v, acc_sh.at[dst_v.at[k]], sem).wait()

        return carry

    lax.fori_loop(0, N_CHUNKS, chunk_body, 0)
    pltpu.make_async_copy(ones_v, acc_sh.at[dst_v.at[0]], sem).wait()

    plsc.subcore_barrier()

    ob = c * N_PAD + s * ROWS_PER_TILE
    for t in range(N_WB):
        pltpu.sync_copy(acc_sh.at[pl.ds(rb + t * 128, 128)], ones_v)
        pltpu.sync_copy(ones_v, cnt_out.at[pl.ds(ob + t * 128, 128)])


BLK = 1024


def _combine_body(relu, p_ref, cnt_ref, x_ref, wl_ref, wr_ref, b_ref, o_ref):
    ssum = p_ref[0] + p_ref[1]
    cnt = cnt_ref[0, :, 0] + cnt_ref[1, :, 0]
    agg = ssum / jnp.clip(cnt, 1.0, None)[:, None]
    out = (jnp.dot(agg, wl_ref[...], preferred_element_type=jnp.float32)
           + jnp.dot(x_ref[...], wr_ref[...], preferred_element_type=jnp.float32)
           + b_ref[0:1, :])
    if relu:
        out = jnp.maximum(out, 0.0)
    o_ref[...] = out


def _combine(p, cnt, x, wlT, wrT, b, relu):
    return pl.pallas_call(
        functools.partial(_combine_body, relu),
        grid=(N_PAD // BLK,),
        in_specs=[
            pl.BlockSpec((2, BLK, D), lambda i: (0, i, 0)),
            pl.BlockSpec((2, BLK, D), lambda i: (0, i, 0)),
            pl.BlockSpec((BLK, D), lambda i: (i, 0)),
            pl.BlockSpec((D, D), lambda i: (0, 0)),
            pl.BlockSpec((D, D), lambda i: (0, 0)),
            pl.BlockSpec((8, D), lambda i: (0, 0)),
        ],
        out_specs=pl.BlockSpec((BLK, D), lambda i: (i, 0)),
        out_shape=jax.ShapeDtypeStruct((N_PAD, D), jnp.float32),
    )(p, cnt, x, wlT, wrT, b)


def kernel(x, edge_index, W1l, b1l, W1r, W2l, b2l, W2r):
    src = edge_index[0].astype(jnp.int32)
    dst = edge_index[1].astype(jnp.int32)

    n_pad_e = E_PAD - N_EDGES
    pad_src = jnp.arange(n_pad_e, dtype=jnp.int32) % N_NODES
    pad_dst = N_NODES + (jnp.arange(n_pad_e, dtype=jnp.int32) % (N_PAD - N_NODES))
    srcr = jnp.concatenate([src, pad_src]).reshape(E_PAD // CHUNK, CHUNK)
    dstr = jnp.concatenate([dst, pad_dst]).reshape(E_PAD // CHUNK, CHUNK)

    zf = jnp.zeros((128, D), jnp.float32)
    ones = jnp.ones((128, D), jnp.float32)
    x_pad = jnp.concatenate([x, jnp.zeros((N_PAD - N_NODES, D), jnp.float32)])

    b1 = jnp.broadcast_to(b1l.reshape(1, D), (8, D))
    b2 = jnp.broadcast_to(b2l.reshape(1, D), (8, D))

    cnt = _degrees(dstr, zf, ones).reshape(2, N_PAD, D)
    p1 = _aggregate(x, srcr, dstr, zf).reshape(2, N_PAD, D)
    h = _combine(p1, cnt, x_pad, W1l.T, W1r.T, b1, relu=True)
    p2 = _aggregate(h, srcr, dstr, zf).reshape(2, N_PAD, D)
    out = _combine(p2, cnt, h, W2l.T, W2r.T, b2, relu=False)
    return out[:N_NODES]


# speedup vs baseline: 10.4268x; 1.0029x over previous
import functools

import jax
import jax.numpy as jnp
from jax import lax
from jax.experimental import pallas as pl
from jax.experimental.pallas import tpu as pltpu
from jax.experimental.pallas import tpu_sc as plsc

N_NODES = 10000
D = 128
N_EDGES = 320000

N_PAD = 10240
E_PAD = 327680
EDGES_PER_TILE = E_PAD // 32
ROWS_PER_TILE = N_PAD // 16
CHUNK = 128
N_CHUNKS = EDGES_PER_TILE // CHUNK
N_WB = ROWS_PER_TILE // 128
PHASES = 2
SLAB = N_CHUNKS // PHASES
PAIRS = SLAB // 2

_sc_mesh = plsc.VectorSubcoreMesh(core_axis_name="c", subcore_axis_name="s")


@functools.partial(
    pl.kernel, mesh=_sc_mesh,
    out_type=jax.ShapeDtypeStruct((2 * N_PAD, D), jnp.float32),
    scratch_types=[
        pltpu.VMEM((SLAB, CHUNK), jnp.int32),
        pltpu.VMEM((SLAB, CHUNK), jnp.int32),
        pltpu.VMEM((CHUNK, D), jnp.float32),
        pltpu.VMEM((CHUNK, D), jnp.float32),
        pltpu.VMEM_SHARED((N_PAD, D), jnp.float32),
        pltpu.SemaphoreType.DMA,
        pltpu.SemaphoreType.DMA,
    ],
)
def _aggregate(x_hbm, src_hbm, dst_hbm, zf, sum_out,
               src_v, dst_v, rows_a, rows_b, acc_sh, sem_a, sem_b):
    c = lax.axis_index("c")
    s = lax.axis_index("s")
    tile = c * 16 + s

    rb = s * ROWS_PER_TILE
    pltpu.sync_copy(zf, rows_a)
    for t in range(N_WB):
        pltpu.sync_copy(rows_a, acc_sh.at[pl.ds(rb + t * 128, 128)])
    plsc.subcore_barrier()

    for p in range(PHASES):
        slab_row = tile * N_CHUNKS + p * SLAB
        pltpu.sync_copy(src_hbm.at[pl.ds(slab_row, SLAB)], src_v)
        pltpu.sync_copy(dst_hbm.at[pl.ds(slab_row, SLAB)], dst_v)

        pltpu.async_copy(x_hbm.at[src_v.at[0]], rows_a, sem_a)

        def pair_body(k2, carry):
            ka = 2 * k2
            pltpu.async_copy(x_hbm.at[src_v.at[ka + 1]], rows_b, sem_b)
            pltpu.make_async_copy(x_hbm.at[src_v.at[ka]], rows_a, sem_a).wait()
            pltpu.sync_copy(rows_a, acc_sh.at[dst_v.at[ka]], add=True)

            @pl.when(k2 < PAIRS - 1)
            def _():
                pltpu.async_copy(x_hbm.at[src_v.at[ka + 2]], rows_a, sem_a)

            pltpu.make_async_copy(x_hbm.at[src_v.at[ka + 1]], rows_b, sem_b).wait()
            pltpu.sync_copy(rows_b, acc_sh.at[dst_v.at[ka + 1]], add=True)
            return carry

        lax.fori_loop(0, PAIRS, pair_body, 0)

    plsc.subcore_barrier()

    ob = c * N_PAD + s * ROWS_PER_TILE
    for t in range(N_WB):
        pltpu.sync_copy(acc_sh.at[pl.ds(rb + t * 128, 128)], rows_a)
        pltpu.sync_copy(rows_a, sum_out.at[pl.ds(ob + t * 128, 128)])


CW = 16


@functools.partial(
    pl.kernel, mesh=_sc_mesh,
    out_type=jax.ShapeDtypeStruct((2 * N_PAD, D), jnp.float32),
    scratch_types=[
        pltpu.VMEM((N_CHUNKS, CHUNK), jnp.int32),
        pltpu.VMEM((CHUNK, D), jnp.float32),
        pltpu.VMEM_SHARED((N_PAD, D), jnp.float32),
        pltpu.SemaphoreType.DMA,
    ],
)
def _degrees(dst_hbm, z16, ones16, cnt_out, dst_v, ones_v, acc_sh, sem):
    c = lax.axis_index("c")
    s = lax.axis_index("s")
    tile = c * 16 + s

    rb = s * ROWS_PER_TILE
    pltpu.sync_copy(z16, ones_v)
    for t in range(N_WB):
        pltpu.sync_copy(ones_v, acc_sh.at[pl.ds(rb + t * 128, 128)])
    pltpu.sync_copy(ones16, ones_v)
    pltpu.sync_copy(dst_hbm.at[pl.ds(tile * N_CHUNKS, N_CHUNKS)], dst_v)
    plsc.subcore_barrier()

    def chunk_body(k, carry):
        pltpu.async_copy(ones_v, acc_sh.at[dst_v.at[k]], sem, add=True)

        @pl.when(k > 0)
        def _():
            pltpu.make_async_copy(ones_v, acc_sh.at[dst_v.at[k]], sem).wait()

        return carry

    lax.fori_loop(0, N_CHUNKS, chunk_body, 0)
    pltpu.make_async_copy(ones_v, acc_sh.at[dst_v.at[0]], sem).wait()

    plsc.subcore_barrier()

    ob = c * N_PAD + s * ROWS_PER_TILE
    for t in range(N_WB):
        pltpu.sync_copy(acc_sh.at[pl.ds(rb + t * 128, 128)], ones_v)
        pltpu.sync_copy(ones_v, cnt_out.at[pl.ds(ob + t * 128, 128)])


BLK = 1024


def _combine_body(relu, p_ref, cnt_ref, x_ref, wl_ref, wr_ref, b_ref, o_ref):
    ssum = p_ref[0] + p_ref[1]
    cnt = cnt_ref[0, :, 0] + cnt_ref[1, :, 0]
    agg = ssum / jnp.clip(cnt, 1.0, None)[:, None]
    out = (jnp.dot(agg, wl_ref[...], preferred_element_type=jnp.float32)
           + jnp.dot(x_ref[...], wr_ref[...], preferred_element_type=jnp.float32)
           + b_ref[0:1, :])
    if relu:
        out = jnp.maximum(out, 0.0)
    o_ref[...] = out


def _combine(p, cnt, x, wlT, wrT, b, relu):
    return pl.pallas_call(
        functools.partial(_combine_body, relu),
        grid=(N_PAD // BLK,),
        in_specs=[
            pl.BlockSpec((2, BLK, D), lambda i: (0, i, 0)),
            pl.BlockSpec((2, BLK, D), lambda i: (0, i, 0)),
            pl.BlockSpec((BLK, D), lambda i: (i, 0)),
            pl.BlockSpec((D, D), lambda i: (0, 0)),
            pl.BlockSpec((D, D), lambda i: (0, 0)),
            pl.BlockSpec((8, D), lambda i: (0, 0)),
        ],
        out_specs=pl.BlockSpec((BLK, D), lambda i: (i, 0)),
        out_shape=jax.ShapeDtypeStruct((N_PAD, D), jnp.float32),
    )(p, cnt, x, wlT, wrT, b)


def kernel(x, edge_index, W1l, b1l, W1r, W2l, b2l, W2r):
    src = edge_index[0].astype(jnp.int32)
    dst = edge_index[1].astype(jnp.int32)

    n_pad_e = E_PAD - N_EDGES
    pad_src = jnp.arange(n_pad_e, dtype=jnp.int32) % N_NODES
    pad_dst = N_NODES + (jnp.arange(n_pad_e, dtype=jnp.int32) % (N_PAD - N_NODES))
    srcr = jnp.concatenate([src, pad_src]).reshape(E_PAD // CHUNK, CHUNK)
    dstr = jnp.concatenate([dst, pad_dst]).reshape(E_PAD // CHUNK, CHUNK)

    zf = jnp.zeros((128, D), jnp.float32)
    x_pad = jnp.concatenate([x, jnp.zeros((N_PAD - N_NODES, D), jnp.float32)])

    b1 = jnp.broadcast_to(b1l.reshape(1, D), (8, D))
    b2 = jnp.broadcast_to(b2l.reshape(1, D), (8, D))

    ones = jnp.ones((CHUNK, D), jnp.float32)
    cnt = _degrees(dstr, zf, ones).reshape(2, N_PAD, D)
    p1 = _aggregate(x, srcr, dstr, zf).reshape(2, N_PAD, D)
    h = _combine(p1, cnt, x_pad, W1l.T, W1r.T, b1, relu=True)
    p2 = _aggregate(h, srcr, dstr, zf).reshape(2, N_PAD, D)
    out = _combine(p2, cnt, h, W2l.T, W2r.T, b2, relu=False)
    return out[:N_NODES]


# speedup vs baseline: 12.7731x; 1.2250x over previous
import functools

import jax
import jax.numpy as jnp
from jax import lax
from jax.experimental import pallas as pl
from jax.experimental.pallas import tpu as pltpu
from jax.experimental.pallas import tpu_sc as plsc

N_NODES = 10000
D = 128
N_EDGES = 320000

N_PAD = 10240
E_PAD = 327680
EDGES_PER_TILE = E_PAD // 32
ROWS_PER_TILE = N_PAD // 16
CHUNK = 128
N_CHUNKS = EDGES_PER_TILE // CHUNK
N_WB = ROWS_PER_TILE // 128
PHASES = 2
SLAB = N_CHUNKS // PHASES
PAIRS = SLAB // 2

_sc_mesh = plsc.VectorSubcoreMesh(core_axis_name="c", subcore_axis_name="s")


@functools.partial(
    pl.kernel, mesh=_sc_mesh,
    out_type=jax.ShapeDtypeStruct((2 * N_PAD, D), jnp.float32),
    scratch_types=[
        pltpu.VMEM((SLAB, CHUNK), jnp.int32),
        pltpu.VMEM((SLAB, CHUNK), jnp.int32),
        pltpu.VMEM((CHUNK, D), jnp.float32),
        pltpu.VMEM((CHUNK, D), jnp.float32),
        pltpu.VMEM_SHARED((N_PAD, D), jnp.float32),
        pltpu.SemaphoreType.DMA,
        pltpu.SemaphoreType.DMA,
    ],
)
def _aggregate(x_hbm, src_hbm, dst_hbm, zf, sum_out,
               src_v, dst_v, rows_a, rows_b, acc_sh, sem_a, sem_b):
    c = lax.axis_index("c")
    s = lax.axis_index("s")
    tile = c * 16 + s

    rb = s * ROWS_PER_TILE
    pltpu.sync_copy(zf, rows_a)
    for t in range(N_WB):
        pltpu.sync_copy(rows_a, acc_sh.at[pl.ds(rb + t * 128, 128)])
    plsc.subcore_barrier()

    for p in range(PHASES):
        slab_row = tile * N_CHUNKS + p * SLAB
        pltpu.sync_copy(src_hbm.at[pl.ds(slab_row, SLAB)], src_v)
        pltpu.sync_copy(dst_hbm.at[pl.ds(slab_row, SLAB)], dst_v)

        pltpu.async_copy(x_hbm.at[src_v.at[0]], rows_a, sem_a)

        def pair_body(k2, carry):
            ka = 2 * k2
            pltpu.async_copy(x_hbm.at[src_v.at[ka + 1]], rows_b, sem_b)
            pltpu.make_async_copy(x_hbm.at[src_v.at[ka]], rows_a, sem_a).wait()
            pltpu.sync_copy(rows_a, acc_sh.at[dst_v.at[ka]], add=True)

            @pl.when(k2 < PAIRS - 1)
            def _():
                pltpu.async_copy(x_hbm.at[src_v.at[ka + 2]], rows_a, sem_a)

            pltpu.make_async_copy(x_hbm.at[src_v.at[ka + 1]], rows_b, sem_b).wait()
            pltpu.sync_copy(rows_b, acc_sh.at[dst_v.at[ka + 1]], add=True)
            return carry

        lax.fori_loop(0, PAIRS, pair_body, 0)

    plsc.subcore_barrier()

    ob = c * N_PAD + s * ROWS_PER_TILE
    for t in range(N_WB):
        pltpu.sync_copy(acc_sh.at[pl.ds(rb + t * 128, 128)], rows_a)
        pltpu.sync_copy(rows_a, sum_out.at[pl.ds(ob + t * 128, 128)])


CROWS = N_PAD // 128


@functools.partial(
    pl.kernel, mesh=_sc_mesh,
    compiler_params=pltpu.CompilerParams(needs_layout_passes=False),
    out_type=jax.ShapeDtypeStruct((2 * CROWS, 128), jnp.float32),
    scratch_types=[
        pltpu.VMEM((EDGES_PER_TILE,), jnp.int32),
        pltpu.VMEM((CROWS, 128), jnp.float32),
        pltpu.VMEM((CROWS,), jnp.int32),
        pltpu.VMEM_SHARED((CROWS, 128), jnp.float32),
        pltpu.SemaphoreType.DMA,
    ],
)
def _degrees(dst_hbm, zf, iota_hbm, cnt_out, dst_v, hist_v, iota_v, acc_sh, sem):
    c = lax.axis_index("c")
    s = lax.axis_index("s")
    tile = c * 16 + s

    pltpu.sync_copy(zf.at[pl.ds(0, CROWS)], hist_v)

    @pl.when(s == 0)
    def _():
        pltpu.sync_copy(zf.at[pl.ds(0, CROWS)], acc_sh)

    pltpu.sync_copy(dst_hbm.at[pl.ds(tile * EDGES_PER_TILE, EDGES_PER_TILE)],
                    dst_v)
    pltpu.sync_copy(iota_hbm, iota_v)
    plsc.subcore_barrier()

    ones16 = jnp.ones((16,), jnp.float32)

    def hbody(i, carry):
        idx = dst_v[pl.ds(i * 16, 16)]
        row = lax.shift_right_logical(idx, 7)
        col = lax.bitwise_and(idx, 127)
        plsc.addupdate_scatter(hist_v, [row, col], ones16)
        return carry

    lax.fori_loop(0, EDGES_PER_TILE // 16, hbody, 0)

    pltpu.sync_copy(hist_v, acc_sh.at[iota_v], add=True)
    plsc.subcore_barrier()

    @pl.when(s == 0)
    def _():
        pltpu.sync_copy(acc_sh, hist_v)
        pltpu.sync_copy(hist_v, cnt_out.at[pl.ds(c * CROWS, CROWS)])


BLK = 1024


def _combine_body(relu, p_ref, cnt_ref, x_ref, wl_ref, wr_ref, b_ref, o_ref):
    ssum = p_ref[0] + p_ref[1]
    cnt = (cnt_ref[0] + cnt_ref[1]).reshape(BLK)
    agg = ssum / jnp.clip(cnt, 1.0, None)[:, None]
    out = (jnp.dot(agg, wl_ref[...], preferred_element_type=jnp.float32)
           + jnp.dot(x_ref[...], wr_ref[...], preferred_element_type=jnp.float32)
           + b_ref[0:1, :])
    if relu:
        out = jnp.maximum(out, 0.0)
    o_ref[...] = out


def _combine(p, cnt, x, wlT, wrT, b, relu):
    return pl.pallas_call(
        functools.partial(_combine_body, relu),
        grid=(N_PAD // BLK,),
        in_specs=[
            pl.BlockSpec((2, BLK, D), lambda i: (0, i, 0)),
            pl.BlockSpec((2, BLK // 128, 128), lambda i: (0, i, 0)),
            pl.BlockSpec((BLK, D), lambda i: (i, 0)),
            pl.BlockSpec((D, D), lambda i: (0, 0)),
            pl.BlockSpec((D, D), lambda i: (0, 0)),
            pl.BlockSpec((8, D), lambda i: (0, 0)),
        ],
        out_specs=pl.BlockSpec((BLK, D), lambda i: (i, 0)),
        out_shape=jax.ShapeDtypeStruct((N_PAD, D), jnp.float32),
    )(p, cnt, x, wlT, wrT, b)


def kernel(x, edge_index, W1l, b1l, W1r, W2l, b2l, W2r):
    src = edge_index[0].astype(jnp.int32)
    dst = edge_index[1].astype(jnp.int32)

    n_pad_e = E_PAD - N_EDGES
    pad_src = jnp.arange(n_pad_e, dtype=jnp.int32) % N_NODES
    pad_dst = N_NODES + (jnp.arange(n_pad_e, dtype=jnp.int32) % (N_PAD - N_NODES))
    srcr = jnp.concatenate([src, pad_src]).reshape(E_PAD // CHUNK, CHUNK)
    dstr = jnp.concatenate([dst, pad_dst]).reshape(E_PAD // CHUNK, CHUNK)

    zf = jnp.zeros((128, D), jnp.float32)
    x_pad = jnp.concatenate([x, jnp.zeros((N_PAD - N_NODES, D), jnp.float32)])

    b1 = jnp.broadcast_to(b1l.reshape(1, D), (8, D))
    b2 = jnp.broadcast_to(b2l.reshape(1, D), (8, D))

    dst_flat = jnp.concatenate([dst, pad_dst])
    iota80 = jnp.arange(CROWS, dtype=jnp.int32)
    cnt = _degrees(dst_flat, zf, iota80).reshape(2, CROWS, 128)
    p1 = _aggregate(x, srcr, dstr, zf).reshape(2, N_PAD, D)
    h = _combine(p1, cnt, x_pad, W1l.T, W1r.T, b1, relu=True)
    p2 = _aggregate(h, srcr, dstr, zf).reshape(2, N_PAD, D)
    out = _combine(p2, cnt, h, W2l.T, W2r.T, b2, relu=False)
    return out[:N_NODES]
